# baseline (speedup 1.0000x reference)
import functools

import jax
import jax.numpy as jnp
from jax.experimental import pallas as pl
from jax.experimental.pallas import tpu as pltpu

D = 1024
H = 2048
E = 8
K = 2
TM = 256


_DNT = (((1,), (1,)), ((), ()))


def _grouped_ffn_kernel(te_ref, xin_ref, w1_ref, w3_ref, w2_ref, sc_ref, out_ref):
    del te_ref
    xb = xin_ref[...].astype(jnp.bfloat16)
    a = jax.lax.dot_general(xb, w1_ref[0].astype(jnp.bfloat16), _DNT,
                            preferred_element_type=jnp.float32)
    b = jax.lax.dot_general(xb, w3_ref[0].astype(jnp.bfloat16), _DNT,
                            preferred_element_type=jnp.float32)
    h = (jax.nn.silu(a) * b).astype(jnp.bfloat16)
    o = jax.lax.dot_general(h, w2_ref[0].astype(jnp.bfloat16), _DNT,
                            preferred_element_type=jnp.float32)
    out_ref[...] = o * sc_ref[...]


def _shared_ffn_kernel(x_ref, w1_ref, w3_ref, w2_ref, out_ref):
    xb = x_ref[...].astype(jnp.bfloat16)
    a = jax.lax.dot_general(xb, w1_ref[...].astype(jnp.bfloat16), _DNT,
                            preferred_element_type=jnp.float32)
    b = jax.lax.dot_general(xb, w3_ref[...].astype(jnp.bfloat16), _DNT,
                            preferred_element_type=jnp.float32)
    h = (jax.nn.silu(a) * b).astype(jnp.bfloat16)
    o = jax.lax.dot_general(h, w2_ref[...].astype(jnp.bfloat16), _DNT,
                            preferred_element_type=jnp.float32)
    out_ref[...] = o


@functools.partial(jax.jit, static_argnames=())
def _run(x, gate_w, w1, w2, w3, sw1, sw2, sw3, expert_bias):
    bs, slen, dim = x.shape
    T = bs * slen
    R = T * K
    NPAD = R + E * TM
    NT = NPAD // TM
    xf = x.reshape(T, D)

    scores = jax.nn.sigmoid(jnp.dot(xf, gate_w.T))
    biased = scores + expert_bias[None, :]
    _, sel = jax.lax.top_k(biased, K)
    top_scores = jnp.take_along_axis(scores, sel, axis=1)
    sel_flat = sel.reshape(-1)
    order = jnp.argsort(sel_flat, stable=True)
    tok_idx = (order // K).astype(jnp.int32)
    expert_sorted = sel_flat[order]
    s_sorted = top_scores.reshape(-1)[order]

    g = jnp.bincount(expert_sorted, length=E)
    starts = jnp.concatenate([jnp.zeros(1, g.dtype), jnp.cumsum(g)[:-1]])
    gpad = ((g + TM - 1) // TM) * TM
    ps = jnp.concatenate([jnp.zeros(1, g.dtype), jnp.cumsum(gpad)[:-1]])
    i = jnp.arange(R)
    p = ps[expert_sorted] + (i - starts[expert_sorted])
    gidx = jnp.zeros((NPAD,), jnp.int32).at[p].set(tok_idx)
    sc = jnp.zeros((NPAD,), jnp.float32).at[p].set(s_sorted)
    tstart = (ps // TM).astype(jnp.int32)
    tids = jnp.arange(NT, dtype=jnp.int32)
    te = jnp.sum(tids[:, None] >= tstart[None, :], axis=1).astype(jnp.int32) - 1

    xin = xf[gidx] * sc[:, None]

    grid_spec = pltpu.PrefetchScalarGridSpec(
        num_scalar_prefetch=1,
        grid=(NT,),
        in_specs=[
            pl.BlockSpec((TM, D), lambda t, te: (t, 0)),
            pl.BlockSpec((1, H, D), lambda t, te: (te[t], 0, 0)),
            pl.BlockSpec((1, H, D), lambda t, te: (te[t], 0, 0)),
            pl.BlockSpec((1, D, H), lambda t, te: (te[t], 0, 0)),
            pl.BlockSpec((TM, 1), lambda t, te: (t, 0)),
        ],
        out_specs=pl.BlockSpec((TM, D), lambda t, te: (t, 0)),
    )
    routed = pl.pallas_call(
        _grouped_ffn_kernel,
        grid_spec=grid_spec,
        out_shape=jax.ShapeDtypeStruct((NPAD, D), jnp.float32),
    )(te, xin, w1, w3, w2, sc[:, None])

    shared = pl.pallas_call(
        _shared_ffn_kernel,
        grid=(T // TM,),
        in_specs=[
            pl.BlockSpec((TM, D), lambda t: (t, 0)),
            pl.BlockSpec((H, D), lambda t: (0, 0)),
            pl.BlockSpec((H, D), lambda t: (0, 0)),
            pl.BlockSpec((D, H), lambda t: (0, 0)),
        ],
        out_specs=pl.BlockSpec((TM, D), lambda t: (t, 0)),
        out_shape=jax.ShapeDtypeStruct((T, D), jnp.float32),
    )(xf, sw1, sw3, sw2)

    out = shared.at[gidx].add(routed)
    return out.reshape(bs, slen, dim)


def kernel(x, gate_w, w1, w2, w3, sw1, sw2, sw3, expert_bias):
    return _run(x, gate_w, w1, w2, w3, sw1, sw2, sw3, expert_bias)


# speedup vs baseline: 1.0952x; 1.0952x over previous
import functools

import jax
import jax.numpy as jnp
from jax.experimental import pallas as pl
from jax.experimental.pallas import tpu as pltpu

D = 1024
H = 2048
E = 8
K = 2
TM = 256


_DNT = (((1,), (1,)), ((), ()))


def _grouped_ffn_kernel(te_ref, xin_ref, w1_ref, w3_ref, w2_ref, sc_ref, out_ref):
    del te_ref
    xb = (xin_ref[...] * sc_ref[...]).astype(jnp.bfloat16)
    a = jax.lax.dot_general(xb, w1_ref[0].astype(jnp.bfloat16), _DNT,
                            preferred_element_type=jnp.float32)
    b = jax.lax.dot_general(xb, w3_ref[0].astype(jnp.bfloat16), _DNT,
                            preferred_element_type=jnp.float32)
    h = (jax.nn.silu(a) * b).astype(jnp.bfloat16)
    o = jax.lax.dot_general(h, w2_ref[0].astype(jnp.bfloat16), _DNT,
                            preferred_element_type=jnp.float32)
    out_ref[...] = o * sc_ref[...]


def _shared_ffn_kernel(x_ref, w1_ref, w3_ref, w2_ref, out_ref):
    xb = x_ref[...].astype(jnp.bfloat16)
    a = jax.lax.dot_general(xb, w1_ref[...].astype(jnp.bfloat16), _DNT,
                            preferred_element_type=jnp.float32)
    b = jax.lax.dot_general(xb, w3_ref[...].astype(jnp.bfloat16), _DNT,
                            preferred_element_type=jnp.float32)
    h = (jax.nn.silu(a) * b).astype(jnp.bfloat16)
    o = jax.lax.dot_general(h, w2_ref[...].astype(jnp.bfloat16), _DNT,
                            preferred_element_type=jnp.float32)
    out_ref[...] = o


@functools.partial(jax.jit, static_argnames=())
def _run(x, gate_w, w1, w2, w3, sw1, sw2, sw3, expert_bias):
    bs, slen, dim = x.shape
    T = bs * slen
    R = T * K
    NPAD = R + E * TM
    NT = NPAD // TM
    xf = x.reshape(T, D)

    scores = jax.nn.sigmoid(jnp.dot(xf, gate_w.T))
    biased = scores + expert_bias[None, :]
    _, sel = jax.lax.top_k(biased, K)
    top_scores = jnp.take_along_axis(scores, sel, axis=1)
    sel_flat = sel.reshape(-1)
    order = jnp.argsort(sel_flat, stable=True)
    tok_idx = (order // K).astype(jnp.int32)
    expert_sorted = sel_flat[order]
    s_sorted = top_scores.reshape(-1)[order]

    g = jnp.bincount(expert_sorted, length=E)
    starts = jnp.concatenate([jnp.zeros(1, g.dtype), jnp.cumsum(g)[:-1]])
    gpad = ((g + TM - 1) // TM) * TM
    ps = jnp.concatenate([jnp.zeros(1, g.dtype), jnp.cumsum(gpad)[:-1]])
    i = jnp.arange(R)
    p = ps[expert_sorted] + (i - starts[expert_sorted])
    gidx = jnp.zeros((NPAD,), jnp.int32).at[p].set(tok_idx)
    sc = jnp.zeros((NPAD,), jnp.float32).at[p].set(s_sorted)
    tstart = (ps // TM).astype(jnp.int32)
    tids = jnp.arange(NT, dtype=jnp.int32)
    te = jnp.sum(tids[:, None] >= tstart[None, :], axis=1).astype(jnp.int32) - 1

    xin = xf[gidx]

    grid_spec = pltpu.PrefetchScalarGridSpec(
        num_scalar_prefetch=1,
        grid=(NT,),
        in_specs=[
            pl.BlockSpec((TM, D), lambda t, te: (t, 0)),
            pl.BlockSpec((1, H, D), lambda t, te: (te[t], 0, 0)),
            pl.BlockSpec((1, H, D), lambda t, te: (te[t], 0, 0)),
            pl.BlockSpec((1, D, H), lambda t, te: (te[t], 0, 0)),
            pl.BlockSpec((TM, 1), lambda t, te: (t, 0)),
        ],
        out_specs=pl.BlockSpec((TM, D), lambda t, te: (t, 0)),
    )
    routed = pl.pallas_call(
        _grouped_ffn_kernel,
        grid_spec=grid_spec,
        out_shape=jax.ShapeDtypeStruct((NPAD, D), jnp.float32),
    )(te, xin, w1, w3, w2, sc[:, None])

    shared = pl.pallas_call(
        _shared_ffn_kernel,
        grid=(T // TM,),
        in_specs=[
            pl.BlockSpec((TM, D), lambda t: (t, 0)),
            pl.BlockSpec((H, D), lambda t: (0, 0)),
            pl.BlockSpec((H, D), lambda t: (0, 0)),
            pl.BlockSpec((D, H), lambda t: (0, 0)),
        ],
        out_specs=pl.BlockSpec((TM, D), lambda t: (t, 0)),
        out_shape=jax.ShapeDtypeStruct((T, D), jnp.float32),
    )(xf, sw1, sw3, sw2)

    q = jnp.zeros((R,), jnp.int32).at[order].set(p.astype(jnp.int32)).reshape(T, K)
    out = shared + routed[q[:, 0]] + routed[q[:, 1]]
    return out.reshape(bs, slen, dim)


def kernel(x, gate_w, w1, w2, w3, sw1, sw2, sw3, expert_bias):
    return _run(x, gate_w, w1, w2, w3, sw1, sw2, sw3, expert_bias)
